# baseline (speedup 1.0000x reference)
import functools

import jax
import jax.numpy as jnp
from jax import lax
from jax.experimental import pallas as pl
from jax.experimental.pallas import tpu as pltpu
from jax.experimental.pallas import tpu_sc as plsc

_B = 4096
_EMB = 64
_LANES = 16
_VOCAB_TILE = 128

_info = plsc.get_sparse_core_info()
_NC = _info.num_cores
_NS = _info.num_subcores
_NW = _NC * _NS
_BPW = _B // _NW

_NCH = 4
_NCHUNKS = _BPW // _NCH
_NBUF = 3

_mesh = plsc.VectorSubcoreMesh(core_axis_name="c", subcore_axis_name="s")


@functools.partial(
    pl.kernel,
    mesh=_mesh,
    out_type=jax.ShapeDtypeStruct((_B,), jnp.float32),
    compiler_params=pltpu.CompilerParams(needs_layout_passes=False),
    scratch_types=[
        pltpu.VMEM((_BPW,), jnp.int32),
        pltpu.VMEM((_NBUF, _NCH, _EMB, _VOCAB_TILE), jnp.float32),
        pltpu.VMEM((_EMB + _LANES,), jnp.float32),
        pltpu.VMEM((_BPW * _LANES,), jnp.float32),
        pltpu.VMEM((_BPW,), jnp.float32),
        pltpu.SemaphoreType.DMA,
    ],
)
def _sc_forward(table_t_hbm, idx_hbm, wb_hbm, out_hbm,
                idx_v, tiles_v, wb_v, par_v, res_v, sem):
    wid = lax.axis_index("s") * _NC + lax.axis_index("c")
    base = wid * _BPW

    pltpu.sync_copy(idx_hbm.at[pl.ds(base, _BPW)], idx_v)
    pltpu.sync_copy(wb_hbm, wb_v)

    lane = lax.iota(jnp.int32, _LANES)
    bias = wb_v[pl.ds(_EMB, _LANES)]
    w_vecs = [wb_v[pl.ds(c * _LANES, _LANES)] for c in range(_EMB // _LANES)]
    zeros16 = jnp.zeros((_LANES,), jnp.int32)

    def chunk_scalars(ch):
        vec = idx_v[pl.ds((ch // _NCH) * _LANES, _LANES)]
        return [vec[(ch % _NCH) * _NCH + q] for q in range(_NCH)]

    def fire(ch, slot):
        hs = []
        for q, i in enumerate(chunk_scalars(ch)):
            cb = pl.multiple_of((i >> 7) << 7, _VOCAB_TILE)
            hs.append(pltpu.async_copy(
                table_t_hbm.at[:, pl.ds(cb, _VOCAB_TILE)],
                tiles_v.at[slot, q], sem))
        return hs

    def process(ch, slot):
        for q, i in enumerate(chunk_scalars(ch)):
            j = ch * _NCH + q
            cvec = zeros16 + (i & (_VOCAB_TILE - 1))
            acc = jnp.zeros((_LANES,), jnp.float32)
            for dg in range(_EMB // _LANES):
                v = plsc.load_gather(tiles_v.at[slot, q],
                                     [lane + dg * _LANES, cvec])
                acc = acc + v * w_vecs[dg]
            par_v[pl.ds(j * _LANES, _LANES)] = acc

    pending = {}
    for ch in range(_NBUF - 1):
        pending[ch] = fire(ch, ch % _NBUF)
    for ch in range(_NCHUNKS):
        nxt = ch + _NBUF - 1
        if nxt < _NCHUNKS:
            pending[nxt] = fire(nxt, nxt % _NBUF)
        for h in pending.pop(ch):
            h.wait()
        process(ch, ch % _NBUF)

    for g in range(_BPW // _LANES):
        acc = bias
        for l in range(_LANES):
            acc = acc + plsc.load_gather(
                par_v, [lane * _LANES + (g * _LANES * _LANES + l)])
        res_v[pl.ds(g * _LANES, _LANES)] = 1.0 / (1.0 + jnp.exp(-acc))

    pltpu.sync_copy(res_v, out_hbm.at[pl.ds(base, _BPW)])


def kernel(x, emb_table, W, b):
    idx = x[:, -1].astype(jnp.int32)
    wb = jnp.concatenate([
        W.reshape(-1).astype(jnp.float32),
        jnp.broadcast_to(b.astype(jnp.float32).reshape(-1)[:1], (_LANES,)),
    ])
    return _sc_forward(emb_table.T, idx, wb)


# speedup vs baseline: 1.0023x; 1.0023x over previous
import functools

import jax
import jax.numpy as jnp
from jax import lax
from jax.experimental import pallas as pl
from jax.experimental.pallas import tpu as pltpu
from jax.experimental.pallas import tpu_sc as plsc

_B = 4096
_L = 200
_EMB = 64
_LANES = 16
_VOCAB_TILE = 128
_XROW = (_L - 1) % 8
_XBASE = (_L - 1) - _XROW

_info = plsc.get_sparse_core_info()
_NC = _info.num_cores
_NS = _info.num_subcores
_NW = _NC * _NS
_BPW = _B // _NW

_NCH = 4
_NCHUNKS = _BPW // _NCH
_NBUF = 2

_mesh = plsc.VectorSubcoreMesh(core_axis_name="c", subcore_axis_name="s")


@functools.partial(
    pl.kernel,
    mesh=_mesh,
    out_type=jax.ShapeDtypeStruct((_B,), jnp.float32),
    compiler_params=pltpu.CompilerParams(needs_layout_passes=False),
    scratch_types=[
        pltpu.VMEM((8, _BPW), jnp.int32),
        pltpu.VMEM((_NBUF, _NCH, _EMB, _VOCAB_TILE), jnp.float32),
        pltpu.VMEM((_EMB,), jnp.float32),
        pltpu.VMEM((1,), jnp.float32),
        pltpu.VMEM((_BPW * _LANES,), jnp.float32),
        pltpu.VMEM((_BPW,), jnp.float32),
        pltpu.SemaphoreType.DMA,
    ],
)
def _sc_forward(table_t_hbm, x_t_hbm, w_hbm, b_hbm, out_hbm,
                xblk_v, tiles_v, w_v, b_v, par_v, res_v, sem):
    wid = lax.axis_index("s") * _NC + lax.axis_index("c")
    base = wid * _BPW

    pltpu.sync_copy(
        x_t_hbm.at[pl.ds(_XBASE, 8), pl.ds(base, _BPW)], xblk_v)
    pltpu.sync_copy(w_hbm, w_v)
    pltpu.sync_copy(b_hbm, b_v)

    lane = lax.iota(jnp.int32, _LANES)
    zeros16 = jnp.zeros((_LANES,), jnp.int32)
    bias = plsc.load_gather(b_v, [zeros16])
    w_vecs = [w_v[pl.ds(c * _LANES, _LANES)] for c in range(_EMB // _LANES)]

    def chunk_scalars(ch):
        vec = xblk_v[_XROW, pl.ds((ch // _NCH) * _LANES, _LANES)]
        return [vec[(ch % _NCH) * _NCH + q] for q in range(_NCH)]

    def fire(ch, slot):
        hs = []
        for q, i in enumerate(chunk_scalars(ch)):
            cb = pl.multiple_of((i >> 7) << 7, _VOCAB_TILE)
            hs.append(pltpu.async_copy(
                table_t_hbm.at[:, pl.ds(cb, _VOCAB_TILE)],
                tiles_v.at[slot, q], sem))
        return hs

    def process(ch, slot):
        for q, i in enumerate(chunk_scalars(ch)):
            j = ch * _NCH + q
            cvec = zeros16 + (i & (_VOCAB_TILE - 1))
            acc = jnp.zeros((_LANES,), jnp.float32)
            for dg in range(_EMB // _LANES):
                v = plsc.load_gather(tiles_v.at[slot, q],
                                     [lane + dg * _LANES, cvec])
                acc = acc + v * w_vecs[dg]
            par_v[pl.ds(j * _LANES, _LANES)] = acc

    pending = {}
    for ch in range(_NBUF - 1):
        pending[ch] = fire(ch, ch % _NBUF)
    for ch in range(_NCHUNKS):
        nxt = ch + _NBUF - 1
        if nxt < _NCHUNKS:
            pending[nxt] = fire(nxt, nxt % _NBUF)
        for h in pending.pop(ch):
            h.wait()
        process(ch, ch % _NBUF)

    for g in range(_BPW // _LANES):
        acc = bias
        for l in range(_LANES):
            acc = acc + plsc.load_gather(
                par_v, [lane * _LANES + (g * _LANES * _LANES + l)])
        res_v[pl.ds(g * _LANES, _LANES)] = 1.0 / (1.0 + jnp.exp(-acc))

    pltpu.sync_copy(res_v, out_hbm.at[pl.ds(base, _BPW)])


def kernel(x, emb_table, W, b):
    return _sc_forward(
        emb_table.T,
        x.astype(jnp.int32).T,
        W.reshape(-1).astype(jnp.float32),
        b.astype(jnp.float32).reshape(-1),
    )


# speedup vs baseline: 1.0069x; 1.0047x over previous
import functools

import jax
import jax.numpy as jnp
from jax import lax
from jax.experimental import pallas as pl
from jax.experimental.pallas import tpu as pltpu
from jax.experimental.pallas import tpu_sc as plsc

_B = 4096
_L = 200
_EMB = 64
_LANES = 16
_VOCAB_TILE = 128
_XROW = (_L - 1) % 8
_XBASE = (_L - 1) - _XROW

_info = plsc.get_sparse_core_info()
_NC = _info.num_cores
_NS = _info.num_subcores
_NW = _NC * _NS
_BPW = _B // _NW

_NCH = 4
_NCHUNKS = _BPW // _NCH
_NBUF = 2

_mesh = plsc.VectorSubcoreMesh(core_axis_name="c", subcore_axis_name="s")


@functools.partial(
    pl.kernel,
    mesh=_mesh,
    out_type=jax.ShapeDtypeStruct((_B,), jnp.float32),
    compiler_params=pltpu.CompilerParams(
        needs_layout_passes=False, skip_device_barrier=True),
    scratch_types=[
        pltpu.VMEM((8, _BPW), jnp.int32),
        pltpu.VMEM((_NBUF, _NCH, _EMB, _VOCAB_TILE), jnp.float32),
        pltpu.VMEM((_EMB,), jnp.float32),
        pltpu.VMEM((1,), jnp.float32),
        pltpu.VMEM((_BPW * _LANES,), jnp.float32),
        pltpu.VMEM((_BPW,), jnp.float32),
        pltpu.SemaphoreType.DMA,
    ],
)
def _sc_forward(table_t_hbm, x_t_hbm, w_hbm, b_hbm, out_hbm,
                xblk_v, tiles_v, w_v, b_v, par_v, res_v, sem):
    wid = lax.axis_index("s") * _NC + lax.axis_index("c")
    base = wid * _BPW

    pltpu.sync_copy(
        x_t_hbm.at[pl.ds(_XBASE, 8), pl.ds(base, _BPW)], xblk_v)
    pltpu.sync_copy(w_hbm, w_v)
    pltpu.sync_copy(b_hbm, b_v)

    lane = lax.iota(jnp.int32, _LANES)
    zeros16 = jnp.zeros((_LANES,), jnp.int32)
    bias = plsc.load_gather(b_v, [zeros16])
    w_vecs = [w_v[pl.ds(c * _LANES, _LANES)] for c in range(_EMB // _LANES)]

    def chunk_scalars(ch):
        vec = xblk_v[_XROW, pl.ds((ch // _NCH) * _LANES, _LANES)]
        return [vec[(ch % _NCH) * _NCH + q] for q in range(_NCH)]

    def fire(ch, slot):
        hs = []
        for q, i in enumerate(chunk_scalars(ch)):
            cb = pl.multiple_of((i >> 7) << 7, _VOCAB_TILE)
            hs.append(pltpu.async_copy(
                table_t_hbm.at[:, pl.ds(cb, _VOCAB_TILE)],
                tiles_v.at[slot, q], sem))
        return hs

    def process(ch, slot):
        for q, i in enumerate(chunk_scalars(ch)):
            j = ch * _NCH + q
            cvec = zeros16 + (i & (_VOCAB_TILE - 1))
            acc = jnp.zeros((_LANES,), jnp.float32)
            for dg in range(_EMB // _LANES):
                v = plsc.load_gather(tiles_v.at[slot, q],
                                     [lane + dg * _LANES, cvec])
                acc = acc + v * w_vecs[dg]
            par_v[pl.ds(j * _LANES, _LANES)] = acc

    pending = {}
    for ch in range(_NBUF - 1):
        pending[ch] = fire(ch, ch % _NBUF)
    for ch in range(_NCHUNKS):
        nxt = ch + _NBUF - 1
        if nxt < _NCHUNKS:
            pending[nxt] = fire(nxt, nxt % _NBUF)
        for h in pending.pop(ch):
            h.wait()
        process(ch, ch % _NBUF)

    for g in range(_BPW // _LANES):
        acc = bias
        for l in range(_LANES):
            acc = acc + plsc.load_gather(
                par_v, [lane * _LANES + (g * _LANES * _LANES + l)])
        res_v[pl.ds(g * _LANES, _LANES)] = 1.0 / (1.0 + jnp.exp(-acc))

    pltpu.sync_copy(res_v, out_hbm.at[pl.ds(base, _BPW)])


def kernel(x, emb_table, W, b):
    return _sc_forward(
        emb_table.T,
        x.astype(jnp.int32).T,
        W.reshape(-1).astype(jnp.float32),
        b.astype(jnp.float32).reshape(-1),
    )
